# baseline (speedup 1.0000x reference)
import jax
import jax.numpy as jnp
from jax.experimental import pallas as pl
from jax.experimental.pallas import tpu as pltpu

_PACK = 4


def _node_body(ht_ref, lt_ref, w_ref, b_ref, out_ref, rhs_scr):
    rhs_scr[0:8, :] = lt_ref[...]
    rhs_scr[8:12, :] = ht_ref[...]
    out_ref[...] = jnp.dot(w_ref[...], rhs_scr[...],
                           preferred_element_type=jnp.float32) + b_ref[...]


def _edge_body(et_ref, es_ref, eb_ref, ed_ref, w_ref, b_ref,
               mu_ref, dev_ref, out_ref, rhs_scr):
    mu = mu_ref[0]
    dev = dev_ref[0]
    d = ed_ref[...] - mu
    ef = jnp.exp(-(d * d) / dev)
    rhs_scr[0:5, :] = et_ref[...]
    rhs_scr[5:6, :] = es_ref[...]
    rhs_scr[6:9, :] = eb_ref[...]
    rhs_scr[9:10, :] = ef
    out_ref[...] = jnp.dot(w_ref[...], rhs_scr[...],
                           preferred_element_type=jnp.float32) + b_ref[...]


def kernel(wh4, wl4, b_h4, w_et4, w_es4, w_eb4, w_ef4, b_e4, ef_mu, ef_dev,
           h, lap, et, es, ed, eb, sign_key):
    H = b_h4.shape[1] // _PACK
    P = wl4.shape[0] // _PACK
    n, ne = h.shape[0], et.shape[0]
    dh, dt, ds, db, dd = (h.shape[1], et.shape[1], es.shape[1], eb.shape[1],
                          ed.shape[1])

    r = jax.random.uniform(jax.random.wrap_key_data(sign_key), (P,),
                           jnp.float32)
    sign = jnp.where(r >= 0.5, 1.0, -1.0).astype(jnp.float32)

    w_node = jnp.concatenate(
        [wl4[:P, :H] * sign[:, None], wh4[:dh, :H]], axis=0).T
    b_h = b_h4[:1, :H].T
    w_edge = jnp.concatenate(
        [w_et4[:dt, :H], w_es4[:ds, :H], w_eb4[:db, :H], w_ef4[:dd, :H]],
        axis=0).T
    b_e = b_e4[:1, :H].T

    ht, lt = h.T, lap.T
    ett, est, ebt, edt = et.T, es.T, eb.T, ed.T

    bw_n, bw_e = 65536, 32768

    out_ht = pl.pallas_call(
        _node_body,
        out_shape=jax.ShapeDtypeStruct((H, n), jnp.float32),
        grid=(pl.cdiv(n, bw_n),),
        in_specs=[
            pl.BlockSpec((dh, bw_n), lambda i: (0, i)),
            pl.BlockSpec((P, bw_n), lambda i: (0, i)),
            pl.BlockSpec((H, P + dh), lambda i: (0, 0)),
            pl.BlockSpec((H, 1), lambda i: (0, 0)),
        ],
        out_specs=pl.BlockSpec((H, bw_n), lambda i: (0, i)),
        scratch_shapes=[pltpu.VMEM((P + dh, bw_n), jnp.float32)],
        compiler_params=pltpu.CompilerParams(dimension_semantics=("parallel",)),
    )(ht, lt, w_node, b_h)

    out_et = pl.pallas_call(
        _edge_body,
        out_shape=jax.ShapeDtypeStruct((H, ne), jnp.float32),
        grid=(pl.cdiv(ne, bw_e),),
        in_specs=[
            pl.BlockSpec((dt, bw_e), lambda i: (0, i)),
            pl.BlockSpec((ds, bw_e), lambda i: (0, i)),
            pl.BlockSpec((db, bw_e), lambda i: (0, i)),
            pl.BlockSpec((dd, bw_e), lambda i: (0, i)),
            pl.BlockSpec((H, dt + ds + db + dd), lambda i: (0, 0)),
            pl.BlockSpec((H, 1), lambda i: (0, 0)),
            pl.BlockSpec(memory_space=pltpu.MemorySpace.SMEM),
            pl.BlockSpec(memory_space=pltpu.MemorySpace.SMEM),
        ],
        out_specs=pl.BlockSpec((H, bw_e), lambda i: (0, i)),
        scratch_shapes=[pltpu.VMEM((dt + ds + db + dd, bw_e), jnp.float32)],
        compiler_params=pltpu.CompilerParams(dimension_semantics=("parallel",)),
    )(ett, est, ebt, edt, w_edge, b_e, ef_mu, ef_dev)

    return out_ht.T, out_et.T, sign.reshape(1, -1)
